# baseline (speedup 1.0000x reference)
import functools

import jax
import jax.numpy as jnp
from jax import lax
from jax.experimental import pallas as pl
from jax.experimental.pallas import tpu as pltpu
from jax.experimental.pallas import tpu_sc as plsc

EMBED = 128
G = 80
NBUF = 8
LAG = 3
NU = 16
UPB = NU // 2
SUPER = NU * G


def _sc_gather(idx_flat, table):
    B = idx_flat.shape[0]
    info = plsc.get_sparse_core_info()
    nw = info.num_cores * info.num_subcores
    b_per_w = B // nw
    n_super = b_per_w // SUPER
    mesh = plsc.VectorSubcoreMesh(core_axis_name="c", subcore_axis_name="s")

    @functools.partial(
        pl.kernel,
        mesh=mesh,
        out_type=jax.ShapeDtypeStruct((B, EMBED), jnp.float32),
        scratch_types=(
            [pltpu.VMEM((UPB * G,), jnp.int32)] * 2
            + [pltpu.VMEM((G, EMBED), jnp.float32)] * NBUF
            + [pltpu.VMEM_SHARED((1001, EMBED), jnp.float32)]
            + [pltpu.SemaphoreType.DMA] * (2 * NBUF + 2)
        ),
    )
    def k(idx_hbm, table_hbm, out_hbm, *scratch):
        idxs = scratch[0:2]
        rows = scratch[2:2 + NBUF]
        table_sh = scratch[2 + NBUF]
        gsem = scratch[3 + NBUF:3 + 2 * NBUF]
        ssem = scratch[3 + 2 * NBUF:3 + 3 * NBUF]
        isem = scratch[3 + 3 * NBUF:]

        sid = lax.axis_index("s")
        wid = sid * info.num_cores + lax.axis_index("c")
        base = wid * b_per_w

        @pl.when(sid == 0)
        def _():
            pltpu.sync_copy(table_hbm, table_sh)

        plsc.subcore_barrier()

        def idx_src(row0):
            return idx_hbm.at[pl.ds(pl.multiple_of(row0, 8), UPB * G)]

        def fire_idx(row0, par):
            return pltpu.async_copy(idx_src(row0), idxs[par], isem[par])

        def idx_wait(par):
            pltpu.make_async_copy(idx_src(base), idxs[par], isem[par]).wait()

        def fire_gather(u, ipar):
            src = table_hbm if u % NBUF == 3 else table_sh
            return pltpu.async_copy(
                src.at[idxs[ipar].at[pl.ds((u % UPB) * G, G)]],
                rows[u % NBUF],
                gsem[u % NBUF],
            )

        def fire_store(u, blk):
            dst = out_hbm.at[pl.ds(pl.multiple_of(blk + u * G, 8), G)]
            return pltpu.async_copy(rows[u % NBUF], dst, ssem[u % NBUF])

        def store_wait(par):
            pltpu.make_async_copy(
                rows[par], out_hbm.at[pl.ds(pl.multiple_of(base, 8), G)],
                ssem[par],
            ).wait()

        fire_idx(base, 0)
        fire_idx(base + UPB * G, 1)

        def body(i, carry):
            blk = pl.multiple_of(base + i * SUPER, 8)
            gh = {}
            sh = {}
            for u in range(NU):
                if u == 0:
                    idx_wait(0)
                if u == UPB:
                    idx_wait(1)
                if u >= NBUF:
                    sh[u - NBUF].wait()
                else:
                    @pl.when(i > 0)
                    def _(par=u % NBUF):
                        store_wait(par)

                gh[u] = fire_gather(u, u // UPB)
                if u >= LAG:
                    gh[u - LAG].wait()
                    sh[u - LAG] = fire_store(u - LAG, blk)
                if u == UPB + LAG - 1:
                    @pl.when(i + 1 < n_super)
                    def _():
                        fire_idx(blk + SUPER, 0)
            for u in range(NU - LAG, NU):
                gh[u].wait()
                sh[u] = fire_store(u, blk)

            @pl.when(i + 1 < n_super)
            def _():
                fire_idx(blk + SUPER + UPB * G, 1)

            return carry

        lax.fori_loop(0, n_super, body, 0)
        for par in range(NBUF):
            store_wait(par)

    return k(idx_flat, table)


def kernel(t, pos_embedding):
    b, h = t.shape
    idx_flat = t.astype(jnp.int32).reshape(b * h)
    out = _sc_gather(idx_flat, pos_embedding)
    return out.reshape(b, h, EMBED)


# speedup vs baseline: 1.1747x; 1.1747x over previous
import functools

import jax
import jax.numpy as jnp
from jax import lax
from jax.experimental import pallas as pl
from jax.experimental.pallas import tpu as pltpu
from jax.experimental.pallas import tpu_sc as plsc

EMBED = 128
G = 80
NBUF = 8
LAG = 3
NU = 16
UPB = NU // 2
SUPER = NU * G


def _sc_gather(idx_flat, table):
    B = idx_flat.shape[0]
    info = plsc.get_sparse_core_info()
    nw = info.num_cores * info.num_subcores
    b_per_w = B // nw
    n_super = b_per_w // SUPER
    mesh = plsc.VectorSubcoreMesh(core_axis_name="c", subcore_axis_name="s")

    @functools.partial(
        pl.kernel,
        mesh=mesh,
        out_type=jax.ShapeDtypeStruct((B, EMBED), jnp.float32),
        scratch_types=(
            [pltpu.VMEM((UPB * G,), jnp.int32)] * 2
            + [pltpu.VMEM((G, EMBED), jnp.float32)] * NBUF
            + [pltpu.VMEM_SHARED((1001, EMBED), jnp.float32)]
            + [pltpu.SemaphoreType.DMA] * (2 * NBUF + 2)
        ),
    )
    def k(idx_hbm, table_hbm, out_hbm, *scratch):
        idxs = scratch[0:2]
        rows = scratch[2:2 + NBUF]
        table_sh = scratch[2 + NBUF]
        gsem = scratch[3 + NBUF:3 + 2 * NBUF]
        ssem = scratch[3 + 2 * NBUF:3 + 3 * NBUF]
        isem = scratch[3 + 3 * NBUF:]

        sid = lax.axis_index("s")
        wid = sid * info.num_cores + lax.axis_index("c")
        base = wid * b_per_w

        @pl.when(sid == 0)
        def _():
            pltpu.sync_copy(table_hbm, table_sh)

        plsc.subcore_barrier()

        def idx_src(row0):
            return idx_hbm.at[pl.ds(pl.multiple_of(row0, 8), UPB * G)]

        def fire_idx(row0, par):
            return pltpu.async_copy(idx_src(row0), idxs[par], isem[par])

        def idx_wait(par):
            pltpu.make_async_copy(idx_src(base), idxs[par], isem[par]).wait()

        def fire_gather(u, ipar):
            src = table_sh
            return pltpu.async_copy(
                src.at[idxs[ipar].at[pl.ds((u % UPB) * G, G)]],
                rows[u % NBUF],
                gsem[u % NBUF],
            )

        def fire_store(u, blk):
            dst = out_hbm.at[pl.ds(pl.multiple_of(blk + u * G, 8), G)]
            return pltpu.async_copy(rows[u % NBUF], dst, ssem[u % NBUF])

        def store_wait(par):
            pltpu.make_async_copy(
                rows[par], out_hbm.at[pl.ds(pl.multiple_of(base, 8), G)],
                ssem[par],
            ).wait()

        fire_idx(base, 0)
        fire_idx(base + UPB * G, 1)

        def body(i, carry):
            blk = pl.multiple_of(base + i * SUPER, 8)
            gh = {}
            sh = {}
            for u in range(NU):
                if u == 0:
                    idx_wait(0)
                if u == UPB:
                    idx_wait(1)
                if u >= NBUF:
                    sh[u - NBUF].wait()
                else:
                    @pl.when(i > 0)
                    def _(par=u % NBUF):
                        store_wait(par)

                gh[u] = fire_gather(u, u // UPB)
                if u >= LAG:
                    gh[u - LAG].wait()
                    sh[u - LAG] = fire_store(u - LAG, blk)
                if u == UPB + LAG - 1:
                    @pl.when(i + 1 < n_super)
                    def _():
                        fire_idx(blk + SUPER, 0)
            for u in range(NU - LAG, NU):
                gh[u].wait()
                sh[u] = fire_store(u, blk)

            @pl.when(i + 1 < n_super)
            def _():
                fire_idx(blk + SUPER + UPB * G, 1)

            return carry

        lax.fori_loop(0, n_super, body, 0)
        for par in range(NBUF):
            store_wait(par)

    return k(idx_flat, table)


def kernel(t, pos_embedding):
    b, h = t.shape
    idx_flat = t.astype(jnp.int32).reshape(b * h)
    out = _sc_gather(idx_flat, pos_embedding)
    return out.reshape(b, h, EMBED)


# speedup vs baseline: 1.1761x; 1.0012x over previous
import functools

import jax
import jax.numpy as jnp
from jax import lax
from jax.experimental import pallas as pl
from jax.experimental.pallas import tpu as pltpu
from jax.experimental.pallas import tpu_sc as plsc

EMBED = 128
G = 80
NBUF = 8
LAG = 4
NU = 16
UPB = NU // 2
SUPER = NU * G


def _sc_gather(idx_flat, table):
    B = idx_flat.shape[0]
    info = plsc.get_sparse_core_info()
    nw = info.num_cores * info.num_subcores
    b_per_w = B // nw
    n_super = b_per_w // SUPER
    mesh = plsc.VectorSubcoreMesh(core_axis_name="c", subcore_axis_name="s")

    @functools.partial(
        pl.kernel,
        mesh=mesh,
        out_type=jax.ShapeDtypeStruct((B, EMBED), jnp.float32),
        scratch_types=(
            [pltpu.VMEM((UPB * G,), jnp.int32)] * 2
            + [pltpu.VMEM((G, EMBED), jnp.float32)] * NBUF
            + [pltpu.VMEM_SHARED((1001, EMBED), jnp.float32)]
            + [pltpu.SemaphoreType.DMA] * (2 * NBUF + 2)
        ),
    )
    def k(idx_hbm, table_hbm, out_hbm, *scratch):
        idxs = scratch[0:2]
        rows = scratch[2:2 + NBUF]
        table_sh = scratch[2 + NBUF]
        gsem = scratch[3 + NBUF:3 + 2 * NBUF]
        ssem = scratch[3 + 2 * NBUF:3 + 3 * NBUF]
        isem = scratch[3 + 3 * NBUF:]

        sid = lax.axis_index("s")
        wid = sid * info.num_cores + lax.axis_index("c")
        base = wid * b_per_w

        @pl.when(sid == 0)
        def _():
            pltpu.sync_copy(table_hbm, table_sh)

        plsc.subcore_barrier()

        def idx_src(row0):
            return idx_hbm.at[pl.ds(pl.multiple_of(row0, 8), UPB * G)]

        def fire_idx(row0, par):
            return pltpu.async_copy(idx_src(row0), idxs[par], isem[par])

        def idx_wait(par):
            pltpu.make_async_copy(idx_src(base), idxs[par], isem[par]).wait()

        def fire_gather(u, ipar):
            src = table_sh
            return pltpu.async_copy(
                src.at[idxs[ipar].at[pl.ds((u % UPB) * G, G)]],
                rows[u % NBUF],
                gsem[u % NBUF],
            )

        def fire_store(u, blk):
            dst = out_hbm.at[pl.ds(pl.multiple_of(blk + u * G, 8), G)]
            return pltpu.async_copy(rows[u % NBUF], dst, ssem[u % NBUF])

        def store_wait(par):
            pltpu.make_async_copy(
                rows[par], out_hbm.at[pl.ds(pl.multiple_of(base, 8), G)],
                ssem[par],
            ).wait()

        fire_idx(base, 0)
        fire_idx(base + UPB * G, 1)

        def body(i, carry):
            blk = pl.multiple_of(base + i * SUPER, 8)
            gh = {}
            sh = {}
            for u in range(NU):
                if u == 0:
                    idx_wait(0)
                if u == UPB:
                    idx_wait(1)
                if u >= NBUF:
                    sh[u - NBUF].wait()
                else:
                    @pl.when(i > 0)
                    def _(par=u % NBUF):
                        store_wait(par)

                gh[u] = fire_gather(u, u // UPB)
                if u >= LAG:
                    gh[u - LAG].wait()
                    sh[u - LAG] = fire_store(u - LAG, blk)
                if u == UPB + LAG - 1:
                    @pl.when(i + 1 < n_super)
                    def _():
                        fire_idx(blk + SUPER, 0)
            for u in range(NU - LAG, NU):
                gh[u].wait()
                sh[u] = fire_store(u, blk)

            @pl.when(i + 1 < n_super)
            def _():
                fire_idx(blk + SUPER + UPB * G, 1)

            return carry

        lax.fori_loop(0, n_super, body, 0)
        for par in range(NBUF):
            store_wait(par)

    return k(idx_flat, table)


def kernel(t, pos_embedding):
    b, h = t.shape
    idx_flat = t.astype(jnp.int32).reshape(b * h)
    out = _sc_gather(idx_flat, pos_embedding)
    return out.reshape(b, h, EMBED)
